# baseline (speedup 1.0000x reference)
import functools

import jax
import jax.numpy as jnp
from jax import lax
from jax.experimental import pallas as pl
from jax.experimental.pallas import tpu as pltpu
from jax.experimental.pallas import tpu_sc as plsc

NW = 32
CHUNK = 128
NSLOT = 2
LANES = 16
T = 0.07
MOMENTUM = 0.5


def _iota16():
    return lax.iota(jnp.int32, LANES)


def _splat16(x):
    return jnp.full((LANES,), x, dtype=jnp.int32)


def _dot_chunk(buf, vv, out_v, c_base, scratch_v, col_idx, lane_base, n_groups):
    ones = jnp.ones((LANES,), dtype=jnp.int32)

    def g_body(g, carry):
        row0 = g * LANES
        for r in range(0, LANES, 4):
            rows = [row0 + r + j for j in range(4)]
            ld = [[buf[rw, pl.ds(p * LANES, LANES)] for p in range(8)]
                  for rw in rows]
            for j in range(4):
                a = [ld[j][p] * vv[p] for p in range(4)]
                for p in range(4, 8):
                    a[p - 4] = a[p - 4] + ld[j][p] * vv[p]
                scratch_v[pl.ds((r + j) * LANES, LANES)] = (a[0] + a[1]) + (a[2] + a[3])
        idxc = lane_base
        tot = plsc.load_gather(scratch_v, [idxc])
        for c in range(1, LANES):
            idxc = idxc + ones
            tot = tot + plsc.load_gather(scratch_v, [idxc])
        out_v[pl.ds(c_base + row0, LANES)] = tot
        return carry

    lax.fori_loop(0, n_groups, g_body, 0)


def _sc_body(v1_hbm, v2_hbm, idx_hbm,
             mem1_hbm, mem2_hbm,
             o1_hbm, o2_hbm,
             *scr):
    idx_all, vb1, vb2 = scr[0:3]
    p = 3
    bufs1 = scr[p:p + NSLOT]; p += NSLOT
    bufs2 = scr[p:p + NSLOT]; p += NSLOT
    obufs1 = scr[p:p + NSLOT]; p += NSLOT
    obufs2 = scr[p:p + NSLOT]; p += NSLOT
    scratch_v = scr[p]; p += 1
    sems1 = scr[p:p + NSLOT]; p += NSLOT
    sems2 = scr[p:p + NSLOT]; p += NSLOT
    osems1 = scr[p:p + NSLOT]; p += NSLOT
    osems2 = scr[p:p + NSLOT]
    NWw, bpw, nc128, _ = idx_hbm.shape
    per128 = 128 // CHUNK
    n_chunks = nc128 * per128
    D = vb1.shape[1]
    wid = lax.axis_index("s") * 2 + lax.axis_index("c")
    inv_t = jnp.float32(1.0 / T)
    cshift = n_chunks.bit_length() - 1
    cmask = n_chunks - 1
    pshift = per128.bit_length() - 1
    pmask = per128 - 1

    col_idx = [_iota16() + p * LANES for p in range(8)]
    lane_base = _iota16() * LANES
    n_groups = CHUNK // LANES

    pltpu.sync_copy(idx_hbm.at[wid], idx_all)
    pltpu.sync_copy(v1_hbm.at[wid], vb1)
    pltpu.sync_copy(v2_hbm.at[wid], vb2)

    n_units = bpw * n_chunks

    def unit_bc(u):
        return lax.shift_right_logical(u, cshift), jnp.bitwise_and(u, cmask)

    def idx_slice(u):
        bl, c = unit_bc(u)
        c2 = lax.shift_right_logical(c, pshift)
        off = jnp.bitwise_and(c, pmask) * CHUNK
        return idx_all.at[bl, c2, pl.ds(off, CHUNK)]

    def start(u, slot):
        isl = idx_slice(u)
        pltpu.async_copy(mem1_hbm.at[isl], bufs1[slot], sems1[slot])
        pltpu.async_copy(mem2_hbm.at[isl], bufs2[slot], sems2[slot])

    def wait1(u, slot):
        pltpu.make_async_copy(mem1_hbm.at[idx_slice(u)], bufs1[slot], sems1[slot]).wait()

    def wait2(u, slot):
        pltpu.make_async_copy(mem2_hbm.at[idx_slice(u)], bufs2[slot], sems2[slot]).wait()

    def owait(par):
        pltpu.make_async_copy(obufs1[par], o1_hbm.at[0, pl.ds(0, CHUNK)], osems1[par]).wait()
        pltpu.make_async_copy(obufs2[par], o2_hbm.at[0, pl.ds(0, CHUNK)], osems2[par]).wait()

    def u_quad(uq, carry2):
        for par in range(NSLOT):
            u = NSLOT * uq + par

            @pl.when(u + NSLOT - 1 < n_units)
            def _():
                start(u + NSLOT - 1, (par + NSLOT - 1) % NSLOT)

            @pl.when(u >= NSLOT)
            def _():
                owait(par)

            bl, c = unit_bc(u)
            vv1 = [vb1[bl, pl.ds(p * LANES, LANES)] * inv_t for p in range(8)]
            vv2 = [vb2[bl, pl.ds(p * LANES, LANES)] * inv_t for p in range(8)]
            wait1(u, par)
            _dot_chunk(bufs1[par], vv2, obufs2[par], 0, scratch_v,
                       col_idx, lane_base, n_groups)
            wait2(u, par)
            _dot_chunk(bufs2[par], vv1, obufs1[par], 0, scratch_v,
                       col_idx, lane_base, n_groups)
            b = wid * bpw + bl
            pltpu.async_copy(obufs1[par], o1_hbm.at[b, pl.ds(c * CHUNK, CHUNK)], osems1[par])
            pltpu.async_copy(obufs2[par], o2_hbm.at[b, pl.ds(c * CHUNK, CHUNK)], osems2[par])
        return carry2

    for u0 in range(NSLOT - 1):
        start(u0, u0)
    lax.fori_loop(0, n_units // NSLOT, u_quad, 0)
    for par in range(NSLOT):
        owait(par)


def _make_sc_call(B, K1, D, N, interpret=False):
    n_chunks = K1 // CHUNK
    bpw = B // NW
    mesh = plsc.VectorSubcoreMesh(core_axis_name="c", subcore_axis_name="s",
                                  num_cores=2, num_subcores=16)
    return pl.kernel(
        _sc_body,
        out_type=(
            jax.ShapeDtypeStruct((B, K1), jnp.float32),
            jax.ShapeDtypeStruct((B, K1), jnp.float32),
        ),
        mesh=mesh,
        scratch_types=(
            [
                pltpu.VMEM((bpw, K1 // 128, 128), jnp.int32),
                pltpu.VMEM((bpw, D), jnp.float32),
                pltpu.VMEM((bpw, D), jnp.float32),
            ]
            + [pltpu.VMEM((CHUNK, D), jnp.float32)] * (2 * NSLOT)
            + [pltpu.VMEM((CHUNK,), jnp.float32)] * (2 * NSLOT)
            + [pltpu.VMEM((LANES * LANES,), jnp.float32)]
            + [pltpu.SemaphoreType.DMA] * (4 * NSLOT)
        ),
        compiler_params=pltpu.CompilerParams(needs_layout_passes=False),
        interpret=interpret,
    )


LAG = 32


def _tc_update_body(y_ref, w_ref, v1_ref, v2_ref,
                    m1_ref, m2_ref, o1_ref, o2_ref,
                    g1_ref, g2_ref, u1_ref, u2_ref,
                    sg1, sg2, sem1, sem2):
    B = y_ref.shape[0]

    def _gwait():
        pltpu.make_async_copy(m1_ref.at[pl.ds(0, 1)], g1_ref.at[pl.ds(0, 1)], sg1).wait()
        pltpu.make_async_copy(m2_ref.at[pl.ds(0, 1)], g2_ref.at[pl.ds(0, 1)], sg2).wait()

    def gbody(i, carry):
        yi = y_ref[i]
        pltpu.make_async_copy(m1_ref.at[pl.ds(yi, 1)], g1_ref.at[pl.ds(i, 1)], sg1).start()
        pltpu.make_async_copy(m2_ref.at[pl.ds(yi, 1)], g2_ref.at[pl.ds(i, 1)], sg2).start()

        @pl.when(i >= LAG)
        def _():
            _gwait()

        return carry

    lax.fori_loop(0, B, gbody, 0)

    def gdrain(i, carry):
        _gwait()
        return carry

    lax.fori_loop(0, min(LAG, B), gdrain, 0)

    t1 = g1_ref[...] * MOMENTUM + v1_ref[...] * (1.0 - MOMENTUM)
    n1 = jnp.sum(t1 * t1, axis=1, keepdims=True)
    u1_ref[...] = t1 / jnp.sqrt(n1)
    t2 = g2_ref[...] * MOMENTUM + v2_ref[...] * (1.0 - MOMENTUM)
    n2 = jnp.sum(t2 * t2, axis=1, keepdims=True)
    u2_ref[...] = t2 / jnp.sqrt(n2)

    def _wait_one():
        pltpu.make_async_copy(u1_ref.at[pl.ds(0, 1)], o1_ref.at[pl.ds(0, 1)], sem1).wait()
        pltpu.make_async_copy(u2_ref.at[pl.ds(0, 1)], o2_ref.at[pl.ds(0, 1)], sem2).wait()

    def sbody(i, carry):
        yi = y_ref[i]
        wi = w_ref[i]
        pltpu.make_async_copy(u1_ref.at[pl.ds(wi, 1)], o1_ref.at[pl.ds(yi, 1)], sem1).start()
        pltpu.make_async_copy(u2_ref.at[pl.ds(wi, 1)], o2_ref.at[pl.ds(yi, 1)], sem2).start()

        @pl.when(i >= LAG)
        def _():
            _wait_one()

        return carry

    lax.fori_loop(0, B, sbody, 0)

    def dbody(i, carry):
        _wait_one()
        return carry

    lax.fori_loop(0, min(LAG, B), dbody, 0)


def _make_tc_update(B, D, N, interpret=False):
    return pl.pallas_call(
        _tc_update_body,
        out_shape=(
            jax.ShapeDtypeStruct((N, D), jnp.float32),
            jax.ShapeDtypeStruct((N, D), jnp.float32),
        ),
        in_specs=[
            pl.BlockSpec(memory_space=pltpu.MemorySpace.SMEM),
            pl.BlockSpec(memory_space=pltpu.MemorySpace.SMEM),
            pl.BlockSpec(memory_space=pltpu.MemorySpace.VMEM),
            pl.BlockSpec(memory_space=pltpu.MemorySpace.VMEM),
            pl.BlockSpec(memory_space=pltpu.MemorySpace.HBM),
            pl.BlockSpec(memory_space=pltpu.MemorySpace.HBM),
        ],
        out_specs=(
            pl.BlockSpec(memory_space=pltpu.MemorySpace.HBM),
            pl.BlockSpec(memory_space=pltpu.MemorySpace.HBM),
        ),
        scratch_shapes=[
            pltpu.VMEM((B, D), jnp.float32),
            pltpu.VMEM((B, D), jnp.float32),
            pltpu.VMEM((B, D), jnp.float32),
            pltpu.VMEM((B, D), jnp.float32),
            pltpu.SemaphoreType.DMA,
            pltpu.SemaphoreType.DMA,
            pltpu.SemaphoreType.DMA,
            pltpu.SemaphoreType.DMA,
        ],
        input_output_aliases={4: 0, 5: 1},
        interpret=interpret,
    )


def _impl(v1, v2, y, idx, memory_v1, memory_v2, interpret=False):
    B, D = v1.shape
    K1 = idx.shape[1]
    N = memory_v1.shape[0]

    iota_b = jnp.arange(B, dtype=jnp.int32)
    eq = y[:, None] == y[None, :]
    w = jnp.max(jnp.where(eq, iota_b[None, :], 0), axis=1).astype(jnp.int32)

    sc_call = _make_sc_call(B, K1, D, N, interpret=interpret)
    out1, out2 = sc_call(
        v1.reshape(NW, B // NW, D), v2.reshape(NW, B // NW, D),
        idx.reshape(NW, B // NW, K1 // 128, 128),
        memory_v1, memory_v2,
    )

    tc_call = _make_tc_update(B, D, N, interpret=interpret)
    new1, new2 = tc_call(y, w, v1, v2, memory_v1, memory_v2)

    return (out1[:, :, None], out2[:, :, None], new1, new2)


def kernel(v1, v2, y, idx, memory_v1, memory_v2):
    return _impl(v1, v2, y, idx, memory_v1, memory_v2, interpret=False)


# speedup vs baseline: 1.0207x; 1.0207x over previous
import functools

import jax
import jax.numpy as jnp
from jax import lax
from jax.experimental import pallas as pl
from jax.experimental.pallas import tpu as pltpu
from jax.experimental.pallas import tpu_sc as plsc

NW = 32
CHUNK = 128
NSLOT = 2
LANES = 16
T = 0.07
MOMENTUM = 0.5


def _iota16():
    return lax.iota(jnp.int32, LANES)


def _splat16(x):
    return jnp.full((LANES,), x, dtype=jnp.int32)


def _dot_chunk(buf, vv, out_v, c_base, scratch_v, col_idx, lane_base, n_groups):
    ones = jnp.ones((LANES,), dtype=jnp.int32)

    def g_body(g, carry):
        row0 = g * LANES
        for r in range(0, LANES, 4):
            rows = [row0 + r + j for j in range(4)]
            ld = [[buf[rw, pl.ds(p * LANES, LANES)] for p in range(8)]
                  for rw in rows]
            for j in range(4):
                a = [ld[j][p] * vv[p] for p in range(4)]
                for p in range(4, 8):
                    a[p - 4] = a[p - 4] + ld[j][p] * vv[p]
                scratch_v[pl.ds((r + j) * LANES, LANES)] = (a[0] + a[1]) + (a[2] + a[3])
        idxc = lane_base
        tot = plsc.load_gather(scratch_v, [idxc])
        for c in range(1, LANES):
            idxc = idxc + ones
            tot = tot + plsc.load_gather(scratch_v, [idxc])
        out_v[pl.ds(c_base + row0, LANES)] = tot
        return carry

    lax.fori_loop(0, n_groups, g_body, 0)


def _sc_body(v1_hbm, v2_hbm, idx_hbm, y_hbm,
             mem1_hbm, mem2_hbm,
             o1_hbm, o2_hbm, g1_hbm, g2_hbm,
             *scr):
    idx_all, vb1, vb2, y_v = scr[0:4]
    p = 4
    bufs1 = scr[p:p + NSLOT]; p += NSLOT
    bufs2 = scr[p:p + NSLOT]; p += NSLOT
    obufs1 = scr[p:p + NSLOT]; p += NSLOT
    obufs2 = scr[p:p + NSLOT]; p += NSLOT
    scratch_v = scr[p]; p += 1
    sems1 = scr[p:p + NSLOT]; p += NSLOT
    sems2 = scr[p:p + NSLOT]; p += NSLOT
    osems1 = scr[p:p + NSLOT]; p += NSLOT
    osems2 = scr[p:p + NSLOT]; p += NSLOT
    gsem = scr[p]
    NWw, bpw, nc128, _ = idx_hbm.shape
    per128 = 128 // CHUNK
    n_chunks = nc128 * per128
    D = vb1.shape[1]
    wid = lax.axis_index("s") * 2 + lax.axis_index("c")
    inv_t = jnp.float32(1.0 / T)
    cshift = n_chunks.bit_length() - 1
    cmask = n_chunks - 1
    pshift = per128.bit_length() - 1
    pmask = per128 - 1

    col_idx = [_iota16() + p * LANES for p in range(8)]
    lane_base = _iota16() * LANES
    n_groups = CHUNK // LANES

    pltpu.sync_copy(idx_hbm.at[wid], idx_all)
    pltpu.sync_copy(v1_hbm.at[wid], vb1)
    pltpu.sync_copy(v2_hbm.at[wid], vb2)

    pltpu.sync_copy(y_hbm.at[wid], y_v)
    gdst1 = bufs1[0].at[pl.ds(0, bpw)]
    gdst2 = bufs2[0].at[pl.ds(0, bpw)]
    pltpu.async_copy(mem1_hbm.at[y_v], gdst1, gsem).wait()
    pltpu.sync_copy(gdst1, g1_hbm.at[pl.ds(wid * bpw, bpw)])
    pltpu.async_copy(mem2_hbm.at[y_v], gdst2, gsem).wait()
    pltpu.sync_copy(gdst2, g2_hbm.at[pl.ds(wid * bpw, bpw)])

    n_units = bpw * n_chunks

    def unit_bc(u):
        return lax.shift_right_logical(u, cshift), jnp.bitwise_and(u, cmask)

    def idx_slice(u):
        bl, c = unit_bc(u)
        c2 = lax.shift_right_logical(c, pshift)
        off = jnp.bitwise_and(c, pmask) * CHUNK
        return idx_all.at[bl, c2, pl.ds(off, CHUNK)]

    def start(u, slot):
        isl = idx_slice(u)
        pltpu.async_copy(mem1_hbm.at[isl], bufs1[slot], sems1[slot])
        pltpu.async_copy(mem2_hbm.at[isl], bufs2[slot], sems2[slot])

    def wait1(u, slot):
        pltpu.make_async_copy(mem1_hbm.at[idx_slice(u)], bufs1[slot], sems1[slot]).wait()

    def wait2(u, slot):
        pltpu.make_async_copy(mem2_hbm.at[idx_slice(u)], bufs2[slot], sems2[slot]).wait()

    def owait(par):
        pltpu.make_async_copy(obufs1[par], o1_hbm.at[0, pl.ds(0, CHUNK)], osems1[par]).wait()
        pltpu.make_async_copy(obufs2[par], o2_hbm.at[0, pl.ds(0, CHUNK)], osems2[par]).wait()

    def u_quad(uq, carry2):
        for par in range(NSLOT):
            u = NSLOT * uq + par

            @pl.when(u + NSLOT - 1 < n_units)
            def _():
                start(u + NSLOT - 1, (par + NSLOT - 1) % NSLOT)

            @pl.when(u >= NSLOT)
            def _():
                owait(par)

            bl, c = unit_bc(u)
            vv1 = [vb1[bl, pl.ds(p * LANES, LANES)] * inv_t for p in range(8)]
            vv2 = [vb2[bl, pl.ds(p * LANES, LANES)] * inv_t for p in range(8)]
            wait1(u, par)
            _dot_chunk(bufs1[par], vv2, obufs2[par], 0, scratch_v,
                       col_idx, lane_base, n_groups)
            wait2(u, par)
            _dot_chunk(bufs2[par], vv1, obufs1[par], 0, scratch_v,
                       col_idx, lane_base, n_groups)
            b = wid * bpw + bl
            pltpu.async_copy(obufs1[par], o1_hbm.at[b, pl.ds(c * CHUNK, CHUNK)], osems1[par])
            pltpu.async_copy(obufs2[par], o2_hbm.at[b, pl.ds(c * CHUNK, CHUNK)], osems2[par])
        return carry2

    for u0 in range(NSLOT - 1):
        start(u0, u0)
    lax.fori_loop(0, n_units // NSLOT, u_quad, 0)
    for par in range(NSLOT):
        owait(par)


def _make_sc_call(B, K1, D, N, interpret=False):
    n_chunks = K1 // CHUNK
    bpw = B // NW
    mesh = plsc.VectorSubcoreMesh(core_axis_name="c", subcore_axis_name="s",
                                  num_cores=2, num_subcores=16)
    return pl.kernel(
        _sc_body,
        out_type=(
            jax.ShapeDtypeStruct((B, K1), jnp.float32),
            jax.ShapeDtypeStruct((B, K1), jnp.float32),
            jax.ShapeDtypeStruct((B, D), jnp.float32),
            jax.ShapeDtypeStruct((B, D), jnp.float32),
        ),
        mesh=mesh,
        scratch_types=(
            [
                pltpu.VMEM((bpw, K1 // 128, 128), jnp.int32),
                pltpu.VMEM((bpw, D), jnp.float32),
                pltpu.VMEM((bpw, D), jnp.float32),
                pltpu.VMEM((bpw,), jnp.int32),
            ]
            + [pltpu.VMEM((CHUNK, D), jnp.float32)] * (2 * NSLOT)
            + [pltpu.VMEM((CHUNK,), jnp.float32)] * (2 * NSLOT)
            + [pltpu.VMEM((LANES * LANES,), jnp.float32)]
            + [pltpu.SemaphoreType.DMA] * (4 * NSLOT + 1)
        ),
        compiler_params=pltpu.CompilerParams(needs_layout_passes=False),
        interpret=interpret,
    )


LAG = 32


def _tc_update_body(y_ref, w_ref, g1_ref, g2_ref, v1_ref, v2_ref,
                    m1_ref, m2_ref, o1_ref, o2_ref,
                    u1_ref, u2_ref, sem1, sem2):
    del m1_ref, m2_ref
    B = y_ref.shape[0]
    t1 = g1_ref[...] * MOMENTUM + v1_ref[...] * (1.0 - MOMENTUM)
    n1 = jnp.sum(t1 * t1, axis=1, keepdims=True)
    u1_ref[...] = t1 / jnp.sqrt(n1)
    t2 = g2_ref[...] * MOMENTUM + v2_ref[...] * (1.0 - MOMENTUM)
    n2 = jnp.sum(t2 * t2, axis=1, keepdims=True)
    u2_ref[...] = t2 / jnp.sqrt(n2)

    def _wait_one():
        pltpu.make_async_copy(u1_ref.at[pl.ds(0, 1)], o1_ref.at[pl.ds(0, 1)], sem1).wait()
        pltpu.make_async_copy(u2_ref.at[pl.ds(0, 1)], o2_ref.at[pl.ds(0, 1)], sem2).wait()

    def sbody(i, carry):
        yi = y_ref[i]
        wi = w_ref[i]
        pltpu.make_async_copy(u1_ref.at[pl.ds(wi, 1)], o1_ref.at[pl.ds(yi, 1)], sem1).start()
        pltpu.make_async_copy(u2_ref.at[pl.ds(wi, 1)], o2_ref.at[pl.ds(yi, 1)], sem2).start()

        @pl.when(i >= LAG)
        def _():
            _wait_one()

        return carry

    lax.fori_loop(0, B, sbody, 0)

    def dbody(i, carry):
        _wait_one()
        return carry

    lax.fori_loop(0, min(LAG, B), dbody, 0)


def _make_tc_update(B, D, N, interpret=False):
    return pl.pallas_call(
        _tc_update_body,
        out_shape=(
            jax.ShapeDtypeStruct((N, D), jnp.float32),
            jax.ShapeDtypeStruct((N, D), jnp.float32),
        ),
        in_specs=[
            pl.BlockSpec(memory_space=pltpu.MemorySpace.SMEM),
            pl.BlockSpec(memory_space=pltpu.MemorySpace.SMEM),
            pl.BlockSpec(memory_space=pltpu.MemorySpace.VMEM),
            pl.BlockSpec(memory_space=pltpu.MemorySpace.VMEM),
            pl.BlockSpec(memory_space=pltpu.MemorySpace.VMEM),
            pl.BlockSpec(memory_space=pltpu.MemorySpace.VMEM),
            pl.BlockSpec(memory_space=pltpu.MemorySpace.HBM),
            pl.BlockSpec(memory_space=pltpu.MemorySpace.HBM),
        ],
        out_specs=(
            pl.BlockSpec(memory_space=pltpu.MemorySpace.HBM),
            pl.BlockSpec(memory_space=pltpu.MemorySpace.HBM),
        ),
        scratch_shapes=[
            pltpu.VMEM((B, D), jnp.float32),
            pltpu.VMEM((B, D), jnp.float32),
            pltpu.SemaphoreType.DMA,
            pltpu.SemaphoreType.DMA,
        ],
        input_output_aliases={6: 0, 7: 1},
        interpret=interpret,
    )


def _impl(v1, v2, y, idx, memory_v1, memory_v2, interpret=False):
    B, D = v1.shape
    K1 = idx.shape[1]
    N = memory_v1.shape[0]

    iota_b = jnp.arange(B, dtype=jnp.int32)
    eq = y[:, None] == y[None, :]
    w = jnp.max(jnp.where(eq, iota_b[None, :], 0), axis=1).astype(jnp.int32)

    sc_call = _make_sc_call(B, K1, D, N, interpret=interpret)
    out1, out2, g1, g2 = sc_call(
        v1.reshape(NW, B // NW, D), v2.reshape(NW, B // NW, D),
        idx.reshape(NW, B // NW, K1 // 128, 128),
        y.reshape(NW, B // NW),
        memory_v1, memory_v2,
    )

    tc_call = _make_tc_update(B, D, N, interpret=interpret)
    new1, new2 = tc_call(y, w, g1, g2, v1, v2, memory_v1, memory_v2)

    return (out1[:, :, None], out2[:, :, None], new1, new2)


def kernel(v1, v2, y, idx, memory_v1, memory_v2):
    return _impl(v1, v2, y, idx, memory_v1, memory_v2, interpret=False)


# speedup vs baseline: 1.0220x; 1.0012x over previous
import jax
import jax.numpy as jnp
from jax import lax
from jax.experimental import pallas as pl
from jax.experimental.pallas import tpu as pltpu
from jax.experimental.pallas import tpu_sc as plsc

NW = 32
CHUNK = 128
NSLOT = 2
LANES = 16
T = 0.07
MOMENTUM = 0.5


def _iota16():
    return lax.iota(jnp.int32, LANES)


def _dot_chunk(buf, vv, out_v, c_base, scratch_v, lane_base, n_groups):
    ones = jnp.ones((LANES,), dtype=jnp.int32)

    def g_body(g, carry):
        row0 = g * LANES
        for r in range(0, LANES, 4):
            rows = [row0 + r + j for j in range(4)]
            ld = [[buf[rw, pl.ds(p * LANES, LANES)] for p in range(8)]
                  for rw in rows]
            for j in range(4):
                a = [ld[j][p] * vv[p] for p in range(4)]
                for p in range(4, 8):
                    a[p - 4] = a[p - 4] + ld[j][p] * vv[p]
                scratch_v[pl.ds((r + j) * LANES, LANES)] = (a[0] + a[1]) + (a[2] + a[3])
        idxc = lane_base
        tot = plsc.load_gather(scratch_v, [idxc])
        for c in range(1, LANES):
            idxc = idxc + ones
            tot = tot + plsc.load_gather(scratch_v, [idxc])
        out_v[pl.ds(c_base + row0, LANES)] = tot
        return carry

    lax.fori_loop(0, n_groups, g_body, 0)


def _sc_body(v1_hbm, v2_hbm, idx_hbm, y_hbm,
             mem1_hbm, mem2_hbm,
             o1_hbm, o2_hbm, g1_hbm, g2_hbm,
             *scr):
    idx_all, vb1, vb2, y_v = scr[0:4]
    p = 4
    bufs1 = scr[p:p + NSLOT]; p += NSLOT
    bufs2 = scr[p:p + NSLOT]; p += NSLOT
    obufs1 = scr[p:p + NSLOT]; p += NSLOT
    obufs2 = scr[p:p + NSLOT]; p += NSLOT
    scratch_v = scr[p]; p += 1
    sems1 = scr[p:p + NSLOT]; p += NSLOT
    sems2 = scr[p:p + NSLOT]; p += NSLOT
    osems1 = scr[p:p + NSLOT]; p += NSLOT
    osems2 = scr[p:p + NSLOT]; p += NSLOT
    gsem = scr[p]
    NWw, bpw, nc128, _ = idx_hbm.shape
    per128 = 128 // CHUNK
    n_chunks = nc128 * per128
    D = vb1.shape[1]
    wid = lax.axis_index("s") * 2 + lax.axis_index("c")
    inv_t = jnp.float32(1.0 / T)
    cshift = n_chunks.bit_length() - 1
    cmask = n_chunks - 1
    pshift = per128.bit_length() - 1
    pmask = per128 - 1

    lane_base = _iota16() * LANES
    n_groups = CHUNK // LANES

    pltpu.sync_copy(idx_hbm.at[wid], idx_all)
    pltpu.sync_copy(v1_hbm.at[wid], vb1)
    pltpu.sync_copy(v2_hbm.at[wid], vb2)

    pltpu.sync_copy(y_hbm.at[wid], y_v)
    gdst1 = bufs1[0].at[pl.ds(0, bpw)]
    gdst2 = bufs2[0].at[pl.ds(0, bpw)]
    pltpu.async_copy(mem1_hbm.at[y_v], gdst1, gsem).wait()
    pltpu.sync_copy(gdst1, g1_hbm.at[pl.ds(wid * bpw, bpw)])
    pltpu.async_copy(mem2_hbm.at[y_v], gdst2, gsem).wait()
    pltpu.sync_copy(gdst2, g2_hbm.at[pl.ds(wid * bpw, bpw)])

    n_units = bpw * n_chunks

    def unit_bc(u):
        return lax.shift_right_logical(u, cshift), jnp.bitwise_and(u, cmask)

    def idx_slice(u):
        bl, c = unit_bc(u)
        c2 = lax.shift_right_logical(c, pshift)
        off = jnp.bitwise_and(c, pmask) * CHUNK
        return idx_all.at[bl, c2, pl.ds(off, CHUNK)]

    def start(u, slot):
        isl = idx_slice(u)
        pltpu.async_copy(mem1_hbm.at[isl], bufs1[slot], sems1[slot])
        pltpu.async_copy(mem2_hbm.at[isl], bufs2[slot], sems2[slot])

    def wait1(u, slot):
        pltpu.make_async_copy(mem1_hbm.at[idx_slice(u)], bufs1[slot], sems1[slot]).wait()

    def wait2(u, slot):
        pltpu.make_async_copy(mem2_hbm.at[idx_slice(u)], bufs2[slot], sems2[slot]).wait()

    def owait(par):
        pltpu.make_async_copy(obufs1[par], o1_hbm.at[0, pl.ds(0, CHUNK)], osems1[par]).wait()
        pltpu.make_async_copy(obufs2[par], o2_hbm.at[0, pl.ds(0, CHUNK)], osems2[par]).wait()

    def u_quad(uq, carry2):
        for par in range(NSLOT):
            u = NSLOT * uq + par

            @pl.when(u + NSLOT - 1 < n_units)
            def _():
                start(u + NSLOT - 1, (par + NSLOT - 1) % NSLOT)

            @pl.when(u >= NSLOT)
            def _():
                owait(par)

            bl, c = unit_bc(u)
            vv1 = [vb1[bl, pl.ds(p * LANES, LANES)] * inv_t for p in range(8)]
            vv2 = [vb2[bl, pl.ds(p * LANES, LANES)] * inv_t for p in range(8)]
            wait1(u, par)
            _dot_chunk(bufs1[par], vv2, obufs2[par], 0, scratch_v,
                       lane_base, n_groups)
            wait2(u, par)
            _dot_chunk(bufs2[par], vv1, obufs1[par], 0, scratch_v,
                       lane_base, n_groups)
            b = wid * bpw + bl
            pltpu.async_copy(obufs1[par], o1_hbm.at[b, pl.ds(c * CHUNK, CHUNK)], osems1[par])
            pltpu.async_copy(obufs2[par], o2_hbm.at[b, pl.ds(c * CHUNK, CHUNK)], osems2[par])
        return carry2

    for u0 in range(NSLOT - 1):
        start(u0, u0)
    lax.fori_loop(0, n_units // NSLOT, u_quad, 0)
    for par in range(NSLOT):
        owait(par)


def _make_sc_call(B, K1, D, N, interpret=False):
    n_chunks = K1 // CHUNK
    bpw = B // NW
    mesh = plsc.VectorSubcoreMesh(core_axis_name="c", subcore_axis_name="s",
                                  num_cores=2, num_subcores=16)
    return pl.kernel(
        _sc_body,
        out_type=(
            jax.ShapeDtypeStruct((B, K1), jnp.float32),
            jax.ShapeDtypeStruct((B, K1), jnp.float32),
            jax.ShapeDtypeStruct((B, D), jnp.float32),
            jax.ShapeDtypeStruct((B, D), jnp.float32),
        ),
        mesh=mesh,
        scratch_types=(
            [
                pltpu.VMEM((bpw, K1 // 128, 128), jnp.int32),
                pltpu.VMEM((bpw, D), jnp.float32),
                pltpu.VMEM((bpw, D), jnp.float32),
                pltpu.VMEM((bpw,), jnp.int32),
            ]
            + [pltpu.VMEM((CHUNK, D), jnp.float32)] * (2 * NSLOT)
            + [pltpu.VMEM((CHUNK,), jnp.float32)] * (2 * NSLOT)
            + [pltpu.VMEM((LANES * LANES,), jnp.float32)]
            + [pltpu.SemaphoreType.DMA] * (4 * NSLOT + 1)
        ),
        compiler_params=pltpu.CompilerParams(needs_layout_passes=False),
        interpret=interpret,
    )


LAG = 32


def _tc_update_body(y_ref, w_ref, g1_ref, g2_ref, v1_ref, v2_ref,
                    m1_ref, m2_ref, o1_ref, o2_ref,
                    u1_ref, u2_ref, sem1, sem2):
    del m1_ref, m2_ref
    B = y_ref.shape[0]
    t1 = g1_ref[...] * MOMENTUM + v1_ref[...] * (1.0 - MOMENTUM)
    n1 = jnp.sum(t1 * t1, axis=1, keepdims=True)
    u1_ref[...] = t1 / jnp.sqrt(n1)
    t2 = g2_ref[...] * MOMENTUM + v2_ref[...] * (1.0 - MOMENTUM)
    n2 = jnp.sum(t2 * t2, axis=1, keepdims=True)
    u2_ref[...] = t2 / jnp.sqrt(n2)

    def _wait_one():
        pltpu.make_async_copy(u1_ref.at[pl.ds(0, 1)], o1_ref.at[pl.ds(0, 1)], sem1).wait()
        pltpu.make_async_copy(u2_ref.at[pl.ds(0, 1)], o2_ref.at[pl.ds(0, 1)], sem2).wait()

    def sbody(i, carry):
        yi = y_ref[i]
        wi = w_ref[i]
        pltpu.make_async_copy(u1_ref.at[pl.ds(wi, 1)], o1_ref.at[pl.ds(yi, 1)], sem1).start()
        pltpu.make_async_copy(u2_ref.at[pl.ds(wi, 1)], o2_ref.at[pl.ds(yi, 1)], sem2).start()

        @pl.when(i >= LAG)
        def _():
            _wait_one()

        return carry

    lax.fori_loop(0, B, sbody, 0)

    def dbody(i, carry):
        _wait_one()
        return carry

    lax.fori_loop(0, min(LAG, B), dbody, 0)


def _make_tc_update(B, D, N, interpret=False):
    return pl.pallas_call(
        _tc_update_body,
        out_shape=(
            jax.ShapeDtypeStruct((N, D), jnp.float32),
            jax.ShapeDtypeStruct((N, D), jnp.float32),
        ),
        in_specs=[
            pl.BlockSpec(memory_space=pltpu.MemorySpace.SMEM),
            pl.BlockSpec(memory_space=pltpu.MemorySpace.SMEM),
            pl.BlockSpec(memory_space=pltpu.MemorySpace.VMEM),
            pl.BlockSpec(memory_space=pltpu.MemorySpace.VMEM),
            pl.BlockSpec(memory_space=pltpu.MemorySpace.VMEM),
            pl.BlockSpec(memory_space=pltpu.MemorySpace.VMEM),
            pl.BlockSpec(memory_space=pltpu.MemorySpace.HBM),
            pl.BlockSpec(memory_space=pltpu.MemorySpace.HBM),
        ],
        out_specs=(
            pl.BlockSpec(memory_space=pltpu.MemorySpace.HBM),
            pl.BlockSpec(memory_space=pltpu.MemorySpace.HBM),
        ),
        scratch_shapes=[
            pltpu.VMEM((B, D), jnp.float32),
            pltpu.VMEM((B, D), jnp.float32),
            pltpu.SemaphoreType.DMA,
            pltpu.SemaphoreType.DMA,
        ],
        input_output_aliases={6: 0, 7: 1},
        interpret=interpret,
    )


def _impl(v1, v2, y, idx, memory_v1, memory_v2, interpret=False):
    B, D = v1.shape
    K1 = idx.shape[1]
    N = memory_v1.shape[0]

    iota_b = jnp.arange(B, dtype=jnp.int32)
    eq = y[:, None] == y[None, :]
    w = jnp.max(jnp.where(eq, iota_b[None, :], 0), axis=1).astype(jnp.int32)

    sc_call = _make_sc_call(B, K1, D, N, interpret=interpret)
    out1, out2, g1, g2 = sc_call(
        v1.reshape(NW, B // NW, D), v2.reshape(NW, B // NW, D),
        idx.reshape(NW, B // NW, K1 // 128, 128),
        y.reshape(NW, B // NW),
        memory_v1, memory_v2,
    )

    tc_call = _make_tc_update(B, D, N, interpret=interpret)
    new1, new2 = tc_call(y, w, g1, g2, v1, v2, memory_v1, memory_v2)

    return (out1[:, :, None], out2[:, :, None], new1, new2)


def kernel(v1, v2, y, idx, memory_v1, memory_v2):
    return _impl(v1, v2, y, idx, memory_v1, memory_v2, interpret=False)
